# baseline (device time: 178655 ns/iter reference)
import jax
import jax.numpy as jnp
from jax import lax
from jax.experimental import pallas as pl
from jax.experimental.pallas import tpu as pltpu

N_DEV = 4
E_LOCAL = 4
N_TOK = 2048
D_IN = 512
D_OUT = 1024
CHUNK = N_TOK // N_DEV


def kernel(x, router_W, route_idx, expert_W):
    my_pos = lax.axis_index("i")

    scores = x @ router_W
    probs = jax.nn.softmax(scores, axis=-1)
    e0 = route_idx[:, 0:1]
    e1 = route_idx[:, 1:2]
    g0 = jnp.take_along_axis(probs, e0, axis=1)
    g1 = jnp.take_along_axis(probs, e1, axis=1)
    gs = g0 + g1
    eids = my_pos * E_LOCAL + jnp.arange(E_LOCAL, dtype=jnp.int32)[None, :]
    w = jnp.where(e0 == eids, g0 / gs, 0.0) + jnp.where(e1 == eids, g1 / gs, 0.0)
    w_pad = jnp.pad(w.astype(jnp.float32), ((0, 0), (0, 128 - E_LOCAL)))

    def body(x_ref, w_ref, ew_ref, out_ref, rbuf, send_sems, recv_sems):
        p = lax.axis_index("i")
        left = lax.rem(p + N_DEV - 1, N_DEV)
        right = lax.rem(p + 1, N_DEV)

        barrier_sem = pltpu.get_barrier_semaphore()
        for nbr in (left, right):
            pl.semaphore_signal(
                barrier_sem, inc=1,
                device_id=(nbr,), device_id_type=pl.DeviceIdType.MESH,
            )
        pl.semaphore_wait(barrier_sem, 2)

        for c in range(N_DEV):
            rows = pl.ds(c * CHUNK, CHUNK)
            xa = x_ref[rows, :]
            acc = jnp.dot(
                xa * w_ref[rows, 0:1], ew_ref[0],
                preferred_element_type=jnp.float32,
            )
            for j in range(1, E_LOCAL):
                acc += jnp.dot(
                    xa * w_ref[rows, j:j + 1], ew_ref[j],
                    preferred_element_type=jnp.float32,
                )
            out_ref[rows, :] = acc

        for h in range(N_DEV - 1):
            sc = lax.rem(p - h + N_DEV, N_DEV)
            rdma = pltpu.make_async_remote_copy(
                src_ref=out_ref.at[pl.ds(sc * CHUNK, CHUNK), :],
                dst_ref=rbuf.at[h],
                send_sem=send_sems.at[h],
                recv_sem=recv_sems.at[h],
                device_id=(right,),
                device_id_type=pl.DeviceIdType.MESH,
            )
            rdma.start()
            rdma.wait()
            rc = lax.rem(p - h - 1 + N_DEV, N_DEV)
            rows = pl.ds(rc * CHUNK, CHUNK)
            out_ref[rows, :] += rbuf[h]

        for h in range(N_DEV - 1):
            sc = lax.rem(p + 1 - h + N_DEV, N_DEV)
            rdma = pltpu.make_async_remote_copy(
                src_ref=out_ref.at[pl.ds(sc * CHUNK, CHUNK), :],
                dst_ref=out_ref.at[pl.ds(sc * CHUNK, CHUNK), :],
                send_sem=send_sems.at[N_DEV - 1 + h],
                recv_sem=recv_sems.at[N_DEV - 1 + h],
                device_id=(right,),
                device_id_type=pl.DeviceIdType.MESH,
            )
            rdma.start()
            rdma.wait()

    n_hops = 2 * (N_DEV - 1)
    return pl.pallas_call(
        body,
        out_shape=jax.ShapeDtypeStruct((N_TOK, D_OUT), jnp.float32),
        in_specs=[
            pl.BlockSpec(memory_space=pltpu.VMEM),
            pl.BlockSpec(memory_space=pltpu.VMEM),
            pl.BlockSpec(memory_space=pltpu.VMEM),
        ],
        out_specs=pl.BlockSpec(memory_space=pltpu.VMEM),
        scratch_shapes=[
            pltpu.VMEM((N_DEV - 1, CHUNK, D_OUT), jnp.float32),
            pltpu.SemaphoreType.DMA((n_hops,)),
            pltpu.SemaphoreType.DMA((n_hops,)),
        ],
        compiler_params=pltpu.CompilerParams(collective_id=0),
    )(x, w_pad, expert_W)


# device time: 174374 ns/iter; 1.0246x vs baseline; 1.0246x over previous
import jax
import jax.numpy as jnp
from jax import lax
from jax.experimental import pallas as pl
from jax.experimental.pallas import tpu as pltpu

N_DEV = 4
N_EXP = 16
E_LOCAL = 4
N_TOK = 2048
D_IN = 512
D_OUT = 1024
CHUNK = N_TOK // N_DEV


def kernel(x, router_W, route_idx, expert_W):
    def body(x_ref, rw_ref, idx_ref, ew_ref, out_ref, rbuf, send_sems, recv_sems):
        p = lax.axis_index("i")
        left = lax.rem(p + N_DEV - 1, N_DEV)
        right = lax.rem(p + 1, N_DEV)

        barrier_sem = pltpu.get_barrier_semaphore()
        for nbr in (left, right):
            pl.semaphore_signal(
                barrier_sem, inc=1,
                device_id=(nbr,), device_id_type=pl.DeviceIdType.MESH,
            )
        pl.semaphore_wait(barrier_sem, 2)

        scores = jnp.dot(x_ref[:, :], rw_ref[:, :],
                         preferred_element_type=jnp.float32)
        m = jnp.max(scores, axis=1, keepdims=True)
        ex = jnp.exp(scores - m)
        probs = ex / jnp.sum(ex, axis=1, keepdims=True)
        idx0 = idx_ref[:, 0:1]
        idx1 = idx_ref[:, 1:2]
        iota = lax.broadcasted_iota(jnp.int32, (N_TOK, N_EXP), 1)
        g0 = jnp.sum(jnp.where(iota == idx0, probs, 0.0), axis=1, keepdims=True)
        g1 = jnp.sum(jnp.where(iota == idx1, probs, 0.0), axis=1, keepdims=True)
        gs = g0 + g1
        g0 = g0 / gs
        g1 = g1 / gs

        for c in range(N_DEV):
            lo, hi = c * CHUNK, (c + 1) * CHUNK
            rows = pl.ds(lo, CHUNK)
            xa = x_ref[rows, :]
            acc = None
            for j in range(E_LOCAL):
                eid = p * E_LOCAL + j
                wj = (jnp.where(idx0[lo:hi] == eid, g0[lo:hi], 0.0)
                      + jnp.where(idx1[lo:hi] == eid, g1[lo:hi], 0.0))
                t = jnp.dot(xa * wj, ew_ref[j],
                            preferred_element_type=jnp.float32)
                acc = t if acc is None else acc + t
            out_ref[rows, :] = acc

        for h in range(N_DEV - 1):
            sc = lax.rem(p - h + N_DEV, N_DEV)
            rdma = pltpu.make_async_remote_copy(
                src_ref=out_ref.at[pl.ds(sc * CHUNK, CHUNK), :],
                dst_ref=rbuf.at[h],
                send_sem=send_sems.at[h],
                recv_sem=recv_sems.at[h],
                device_id=(right,),
                device_id_type=pl.DeviceIdType.MESH,
            )
            rdma.start()
            rdma.wait()
            rc = lax.rem(p - h - 1 + N_DEV, N_DEV)
            rows = pl.ds(rc * CHUNK, CHUNK)
            out_ref[rows, :] += rbuf[h]

        for h in range(N_DEV - 1):
            sc = lax.rem(p + 1 - h + N_DEV, N_DEV)
            rdma = pltpu.make_async_remote_copy(
                src_ref=out_ref.at[pl.ds(sc * CHUNK, CHUNK), :],
                dst_ref=out_ref.at[pl.ds(sc * CHUNK, CHUNK), :],
                send_sem=send_sems.at[N_DEV - 1 + h],
                recv_sem=recv_sems.at[N_DEV - 1 + h],
                device_id=(right,),
                device_id_type=pl.DeviceIdType.MESH,
            )
            rdma.start()
            rdma.wait()

    n_hops = 2 * (N_DEV - 1)
    return pl.pallas_call(
        body,
        out_shape=jax.ShapeDtypeStruct((N_TOK, D_OUT), jnp.float32),
        in_specs=[
            pl.BlockSpec(memory_space=pltpu.VMEM),
            pl.BlockSpec(memory_space=pltpu.VMEM),
            pl.BlockSpec(memory_space=pltpu.VMEM),
            pl.BlockSpec(memory_space=pltpu.VMEM),
        ],
        out_specs=pl.BlockSpec(memory_space=pltpu.VMEM),
        scratch_shapes=[
            pltpu.VMEM((N_DEV - 1, CHUNK, D_OUT), jnp.float32),
            pltpu.SemaphoreType.DMA((n_hops,)),
            pltpu.SemaphoreType.DMA((n_hops,)),
        ],
        compiler_params=pltpu.CompilerParams(collective_id=0),
    )(x, router_W, route_idx, expert_W)


# device time: 107169 ns/iter; 1.6670x vs baseline; 1.6271x over previous
import jax
import jax.numpy as jnp
from jax import lax
from jax.experimental import pallas as pl
from jax.experimental.pallas import tpu as pltpu

N_DEV = 4
N_EXP = 16
E_LOCAL = 4
N_TOK = 2048
D_IN = 512
D_OUT = 1024
CHUNK = N_TOK // N_DEV
HALF = D_OUT // 2


def kernel(x, router_W, route_idx, expert_W):
    def body(x_ref, rw_ref, idx_ref, ew_ref, out_ref,
             rbuf_cw, rbuf_ccw, ss_cw, rs_cw, ss_ccw, rs_ccw):
        p = lax.axis_index("i")
        left = lax.rem(p + N_DEV - 1, N_DEV)
        right = lax.rem(p + 1, N_DEV)

        barrier_sem = pltpu.get_barrier_semaphore()
        for nbr in (left, right):
            pl.semaphore_signal(
                barrier_sem, inc=1,
                device_id=(nbr,), device_id_type=pl.DeviceIdType.MESH,
            )
        pl.semaphore_wait(barrier_sem, 2)

        scores = jnp.dot(x_ref[:, :], rw_ref[:, :],
                         preferred_element_type=jnp.float32)
        m = jnp.max(scores, axis=1, keepdims=True)
        ex = jnp.exp(scores - m)
        probs = ex / jnp.sum(ex, axis=1, keepdims=True)
        idx0 = idx_ref[:, 0:1]
        idx1 = idx_ref[:, 1:2]
        iota = lax.broadcasted_iota(jnp.int32, (N_TOK, N_EXP), 1)
        g0 = jnp.sum(jnp.where(iota == idx0, probs, 0.0), axis=1, keepdims=True)
        g1 = jnp.sum(jnp.where(iota == idx1, probs, 0.0), axis=1, keepdims=True)
        gs = g0 + g1
        g0 = g0 / gs
        g1 = g1 / gs

        for c in range(N_DEV):
            lo, hi = c * CHUNK, (c + 1) * CHUNK
            rows = pl.ds(lo, CHUNK)
            xa = x_ref[rows, :]
            acc = None
            for j in range(E_LOCAL):
                eid = p * E_LOCAL + j
                wj = (jnp.where(idx0[lo:hi] == eid, g0[lo:hi], 0.0)
                      + jnp.where(idx1[lo:hi] == eid, g1[lo:hi], 0.0))
                t = jnp.dot(xa * wj, ew_ref[j],
                            preferred_element_type=jnp.float32)
                acc = t if acc is None else acc + t
            out_ref[rows, :] = acc

        def cw_copy(src_chunk, dst_rbuf_slot, dst_chunk, h):
            if dst_rbuf_slot is not None:
                dst = rbuf_cw.at[dst_rbuf_slot]
            else:
                dst = out_ref.at[pl.ds(dst_chunk * CHUNK, CHUNK), pl.ds(0, HALF)]
            return pltpu.make_async_remote_copy(
                src_ref=out_ref.at[pl.ds(src_chunk * CHUNK, CHUNK), pl.ds(0, HALF)],
                dst_ref=dst,
                send_sem=ss_cw.at[h], recv_sem=rs_cw.at[h],
                device_id=(right,), device_id_type=pl.DeviceIdType.MESH,
            )

        def ccw_copy(src_chunk, dst_rbuf_slot, dst_chunk, h):
            if dst_rbuf_slot is not None:
                dst = rbuf_ccw.at[dst_rbuf_slot]
            else:
                dst = out_ref.at[pl.ds(dst_chunk * CHUNK, CHUNK), pl.ds(HALF, HALF)]
            return pltpu.make_async_remote_copy(
                src_ref=out_ref.at[pl.ds(src_chunk * CHUNK, CHUNK), pl.ds(HALF, HALF)],
                dst_ref=dst,
                send_sem=ss_ccw.at[h], recv_sem=rs_ccw.at[h],
                device_id=(left,), device_id_type=pl.DeviceIdType.MESH,
            )

        for h in range(N_DEV - 1):
            sc_cw = lax.rem(p - h + N_DEV, N_DEV)
            sc_ccw = lax.rem(p + h, N_DEV)
            cw = cw_copy(sc_cw, h, None, h)
            ccw = ccw_copy(sc_ccw, h, None, h)
            cw.start()
            ccw.start()
            cw.wait()
            ccw.wait()
            rc_cw = lax.rem(p - h - 1 + N_DEV, N_DEV)
            rc_ccw = lax.rem(p + h + 1, N_DEV)
            out_ref[pl.ds(rc_cw * CHUNK, CHUNK), 0:HALF] += rbuf_cw[h]
            out_ref[pl.ds(rc_ccw * CHUNK, CHUNK), HALF:D_OUT] += rbuf_ccw[h]

        for h in range(N_DEV - 1):
            sc_cw = lax.rem(p + 1 - h + N_DEV, N_DEV)
            sc_ccw = lax.rem(p - 1 + h + N_DEV, N_DEV)
            cw = cw_copy(sc_cw, None, sc_cw, N_DEV - 1 + h)
            ccw = ccw_copy(sc_ccw, None, sc_ccw, N_DEV - 1 + h)
            cw.start()
            ccw.start()
            cw.wait()
            ccw.wait()

    n_hops = 2 * (N_DEV - 1)
    return pl.pallas_call(
        body,
        out_shape=jax.ShapeDtypeStruct((N_TOK, D_OUT), jnp.float32),
        in_specs=[
            pl.BlockSpec(memory_space=pltpu.VMEM),
            pl.BlockSpec(memory_space=pltpu.VMEM),
            pl.BlockSpec(memory_space=pltpu.VMEM),
            pl.BlockSpec(memory_space=pltpu.VMEM),
        ],
        out_specs=pl.BlockSpec(memory_space=pltpu.VMEM),
        scratch_shapes=[
            pltpu.VMEM((N_DEV - 1, CHUNK, HALF), jnp.float32),
            pltpu.VMEM((N_DEV - 1, CHUNK, HALF), jnp.float32),
            pltpu.SemaphoreType.DMA((n_hops,)),
            pltpu.SemaphoreType.DMA((n_hops,)),
            pltpu.SemaphoreType.DMA((n_hops,)),
            pltpu.SemaphoreType.DMA((n_hops,)),
        ],
        compiler_params=pltpu.CompilerParams(collective_id=0),
    )(x, router_W, route_idx, expert_W)


# device time: 104318 ns/iter; 1.7126x vs baseline; 1.0273x over previous
import jax
import jax.numpy as jnp
from jax import lax
from jax.experimental import pallas as pl
from jax.experimental.pallas import tpu as pltpu

N_DEV = 4
N_EXP = 16
E_LOCAL = 4
N_TOK = 2048
D_IN = 512
D_OUT = 1024
CHUNK = N_TOK // N_DEV
HALF = D_OUT // 2


def kernel(x, router_W, route_idx, expert_W):
    def body(x_ref, rw_ref, idx_ref, ew_ref, out_ref,
             rbuf_cw, rbuf_ccw, w_ref, ss_cw, rs_cw, ss_ccw, rs_ccw):
        p = lax.axis_index("i")
        left = lax.rem(p + N_DEV - 1, N_DEV)
        right = lax.rem(p + 1, N_DEV)

        barrier_sem = pltpu.get_barrier_semaphore()
        for nbr in (left, right):
            pl.semaphore_signal(
                barrier_sem, inc=1,
                device_id=(nbr,), device_id_type=pl.DeviceIdType.MESH,
            )
        pl.semaphore_wait(barrier_sem, 2)

        scores = jnp.dot(x_ref[:, :], rw_ref[:, :],
                         preferred_element_type=jnp.float32)
        m = jnp.max(scores, axis=1, keepdims=True)
        ex = jnp.exp(scores - m)
        probs = ex / jnp.sum(ex, axis=1, keepdims=True)
        idx0 = idx_ref[:, 0:1]
        idx1 = idx_ref[:, 1:2]
        iota = lax.broadcasted_iota(jnp.int32, (N_TOK, N_EXP), 1)
        g0 = jnp.sum(jnp.where(iota == idx0, probs, 0.0), axis=1, keepdims=True)
        g1 = jnp.sum(jnp.where(iota == idx1, probs, 0.0), axis=1, keepdims=True)
        gs = g0 + g1
        g0 = g0 / gs
        g1 = g1 / gs
        for j in range(E_LOCAL):
            eid = p * E_LOCAL + j
            w_ref[:, j:j + 1] = (jnp.where(idx0 == eid, g0, 0.0)
                                 + jnp.where(idx1 == eid, g1, 0.0))

        def cw_copy(src_chunk, dst_rbuf_slot, dst_chunk, h):
            if dst_rbuf_slot is not None:
                dst = rbuf_cw.at[dst_rbuf_slot]
            else:
                dst = out_ref.at[pl.ds(dst_chunk * CHUNK, CHUNK), pl.ds(0, HALF)]
            return pltpu.make_async_remote_copy(
                src_ref=out_ref.at[pl.ds(src_chunk * CHUNK, CHUNK), pl.ds(0, HALF)],
                dst_ref=dst,
                send_sem=ss_cw.at[h], recv_sem=rs_cw.at[h],
                device_id=(right,), device_id_type=pl.DeviceIdType.MESH,
            )

        def ccw_copy(src_chunk, dst_rbuf_slot, dst_chunk, h):
            if dst_rbuf_slot is not None:
                dst = rbuf_ccw.at[dst_rbuf_slot]
            else:
                dst = out_ref.at[pl.ds(dst_chunk * CHUNK, CHUNK), pl.ds(HALF, HALF)]
            return pltpu.make_async_remote_copy(
                src_ref=out_ref.at[pl.ds(src_chunk * CHUNK, CHUNK), pl.ds(HALF, HALF)],
                dst_ref=dst,
                send_sem=ss_ccw.at[h], recv_sem=rs_ccw.at[h],
                device_id=(left,), device_id_type=pl.DeviceIdType.MESH,
            )

        hop0 = [None, None]
        for o in range(N_DEV):
            c = lax.rem(p + o, N_DEV)
            rows = pl.ds(c * CHUNK, CHUNK)
            xa = x_ref[rows, :]
            acc = None
            for j in range(E_LOCAL):
                t = jnp.dot(xa * w_ref[rows, j:j + 1], ew_ref[j],
                            preferred_element_type=jnp.float32)
                acc = t if acc is None else acc + t
            out_ref[rows, :] = acc
            if o == 0:
                hop0[0] = cw_copy(c, 0, None, 0)
                hop0[1] = ccw_copy(c, 0, None, 0)
                hop0[0].start()
                hop0[1].start()

        for h in range(N_DEV - 1):
            if h == 0:
                cw, ccw = hop0
            else:
                sc_cw = lax.rem(p - h + N_DEV, N_DEV)
                sc_ccw = lax.rem(p + h, N_DEV)
                cw = cw_copy(sc_cw, h, None, h)
                ccw = ccw_copy(sc_ccw, h, None, h)
                cw.start()
                ccw.start()
            cw.wait()
            ccw.wait()
            rc_cw = lax.rem(p - h - 1 + N_DEV, N_DEV)
            rc_ccw = lax.rem(p + h + 1, N_DEV)
            out_ref[pl.ds(rc_cw * CHUNK, CHUNK), 0:HALF] += rbuf_cw[h]
            out_ref[pl.ds(rc_ccw * CHUNK, CHUNK), HALF:D_OUT] += rbuf_ccw[h]

        for h in range(N_DEV - 1):
            sc_cw = lax.rem(p + 1 - h + N_DEV, N_DEV)
            sc_ccw = lax.rem(p - 1 + h + N_DEV, N_DEV)
            cw = cw_copy(sc_cw, None, sc_cw, N_DEV - 1 + h)
            ccw = ccw_copy(sc_ccw, None, sc_ccw, N_DEV - 1 + h)
            cw.start()
            ccw.start()
            cw.wait()
            ccw.wait()

    n_hops = 2 * (N_DEV - 1)
    return pl.pallas_call(
        body,
        out_shape=jax.ShapeDtypeStruct((N_TOK, D_OUT), jnp.float32),
        in_specs=[
            pl.BlockSpec(memory_space=pltpu.VMEM),
            pl.BlockSpec(memory_space=pltpu.VMEM),
            pl.BlockSpec(memory_space=pltpu.VMEM),
            pl.BlockSpec(memory_space=pltpu.VMEM),
        ],
        out_specs=pl.BlockSpec(memory_space=pltpu.VMEM),
        scratch_shapes=[
            pltpu.VMEM((N_DEV - 1, CHUNK, HALF), jnp.float32),
            pltpu.VMEM((N_DEV - 1, CHUNK, HALF), jnp.float32),
            pltpu.VMEM((N_TOK, E_LOCAL), jnp.float32),
            pltpu.SemaphoreType.DMA((n_hops,)),
            pltpu.SemaphoreType.DMA((n_hops,)),
            pltpu.SemaphoreType.DMA((n_hops,)),
            pltpu.SemaphoreType.DMA((n_hops,)),
        ],
        compiler_params=pltpu.CompilerParams(collective_id=0),
    )(x, router_W, route_idx, expert_W)


# device time: 72806 ns/iter; 2.4538x vs baseline; 1.4328x over previous
import jax
import jax.numpy as jnp
from jax import lax
from jax.experimental import pallas as pl
from jax.experimental.pallas import tpu as pltpu

N_DEV = 4
N_EXP = 16
E_LOCAL = 4
N_TOK = 2048
D_IN = 512
D_OUT = 1024
CHUNK = N_TOK // N_DEV
HALF = D_OUT // 2
BF16 = jnp.bfloat16
F32 = jnp.float32


def kernel(x, router_W, route_idx, expert_W):
    def body(x_ref, rw_ref, idx_ref, ew_ref, out_ref,
             sbuf_cw, rbuf_cw, agbuf_cw, sbuf_ccw, rbuf_ccw, agbuf_ccw,
             w_ref, ss_cw, rs_cw, ss_ccw, rs_ccw):
        p = lax.axis_index("i")
        left = lax.rem(p + N_DEV - 1, N_DEV)
        right = lax.rem(p + 1, N_DEV)

        barrier_sem = pltpu.get_barrier_semaphore()
        for nbr in (left, right):
            pl.semaphore_signal(
                barrier_sem, inc=1,
                device_id=(nbr,), device_id_type=pl.DeviceIdType.MESH,
            )
        pl.semaphore_wait(barrier_sem, 2)

        scores = jnp.dot(x_ref[:, :], rw_ref[:, :],
                         preferred_element_type=F32)
        m = jnp.max(scores, axis=1, keepdims=True)
        ex = jnp.exp(scores - m)
        probs = ex / jnp.sum(ex, axis=1, keepdims=True)
        idx0 = idx_ref[:, 0:1]
        idx1 = idx_ref[:, 1:2]
        iota = lax.broadcasted_iota(jnp.int32, (N_TOK, N_EXP), 1)
        g0 = jnp.sum(jnp.where(iota == idx0, probs, 0.0), axis=1, keepdims=True)
        g1 = jnp.sum(jnp.where(iota == idx1, probs, 0.0), axis=1, keepdims=True)
        gs = g0 + g1
        g0 = g0 / gs
        g1 = g1 / gs
        for j in range(E_LOCAL):
            eid = p * E_LOCAL + j
            w_ref[:, j:j + 1] = (jnp.where(idx0 == eid, g0, 0.0)
                                 + jnp.where(idx1 == eid, g1, 0.0))

        def rdma(src, dst, ss, rs, h, dev):
            return pltpu.make_async_remote_copy(
                src_ref=src, dst_ref=dst, send_sem=ss.at[h], recv_sem=rs.at[h],
                device_id=(dev,), device_id_type=pl.DeviceIdType.MESH,
            )

        hop0 = [None, None]
        for o in range(N_DEV):
            c = lax.rem(p + o, N_DEV)
            rows = pl.ds(c * CHUNK, CHUNK)
            xa = x_ref[rows, :]
            acc = None
            for j in range(E_LOCAL):
                t = jnp.dot(xa * w_ref[rows, j:j + 1], ew_ref[j],
                            preferred_element_type=F32)
                acc = t if acc is None else acc + t
            if o == 0:
                sbuf_cw[0] = acc[:, 0:HALF].astype(BF16)
                sbuf_ccw[0] = acc[:, HALF:D_OUT].astype(BF16)
                hop0[0] = rdma(sbuf_cw.at[0], rbuf_cw.at[0], ss_cw, rs_cw, 0, right)
                hop0[1] = rdma(sbuf_ccw.at[0], rbuf_ccw.at[0], ss_ccw, rs_ccw, 0, left)
                hop0[0].start()
                hop0[1].start()
            else:
                out_ref[rows, :] = acc

        for h in range(N_DEV - 1):
            if h == 0:
                cw, ccw = hop0
            else:
                cw = rdma(sbuf_cw.at[h], rbuf_cw.at[h], ss_cw, rs_cw, h, right)
                ccw = rdma(sbuf_ccw.at[h], rbuf_ccw.at[h], ss_ccw, rs_ccw, h, left)
                cw.start()
                ccw.start()
            cw.wait()
            ccw.wait()
            rc_cw = lax.rem(p - h - 1 + N_DEV, N_DEV)
            rc_ccw = lax.rem(p + h + 1, N_DEV)
            s_cw = out_ref[pl.ds(rc_cw * CHUNK, CHUNK), 0:HALF] \
                + rbuf_cw[h].astype(F32)
            s_ccw = out_ref[pl.ds(rc_ccw * CHUNK, CHUNK), HALF:D_OUT] \
                + rbuf_ccw[h].astype(F32)
            out_ref[pl.ds(rc_cw * CHUNK, CHUNK), 0:HALF] = s_cw
            out_ref[pl.ds(rc_ccw * CHUNK, CHUNK), HALF:D_OUT] = s_ccw
            if h < N_DEV - 2:
                sbuf_cw[h + 1] = s_cw.astype(BF16)
                sbuf_ccw[h + 1] = s_ccw.astype(BF16)
            else:
                agbuf_cw[0] = s_cw.astype(BF16)
                agbuf_ccw[0] = s_ccw.astype(BF16)

        for h in range(N_DEV - 1):
            cw = rdma(agbuf_cw.at[h], agbuf_cw.at[h + 1],
                      ss_cw, rs_cw, N_DEV - 1 + h, right)
            ccw = rdma(agbuf_ccw.at[h], agbuf_ccw.at[h + 1],
                       ss_ccw, rs_ccw, N_DEV - 1 + h, left)
            cw.start()
            ccw.start()
            cw.wait()
            ccw.wait()
            rc_cw = lax.rem(p - h + N_DEV, N_DEV)
            rc_ccw = lax.rem(p + h, N_DEV)
            out_ref[pl.ds(rc_cw * CHUNK, CHUNK), 0:HALF] = \
                agbuf_cw[h + 1].astype(F32)
            out_ref[pl.ds(rc_ccw * CHUNK, CHUNK), HALF:D_OUT] = \
                agbuf_ccw[h + 1].astype(F32)

    n_hops = 2 * (N_DEV - 1)
    return pl.pallas_call(
        body,
        out_shape=jax.ShapeDtypeStruct((N_TOK, D_OUT), F32),
        in_specs=[
            pl.BlockSpec(memory_space=pltpu.VMEM),
            pl.BlockSpec(memory_space=pltpu.VMEM),
            pl.BlockSpec(memory_space=pltpu.VMEM),
            pl.BlockSpec(memory_space=pltpu.VMEM),
        ],
        out_specs=pl.BlockSpec(memory_space=pltpu.VMEM),
        scratch_shapes=[
            pltpu.VMEM((N_DEV - 1, CHUNK, HALF), BF16),
            pltpu.VMEM((N_DEV - 1, CHUNK, HALF), BF16),
            pltpu.VMEM((N_DEV, CHUNK, HALF), BF16),
            pltpu.VMEM((N_DEV - 1, CHUNK, HALF), BF16),
            pltpu.VMEM((N_DEV - 1, CHUNK, HALF), BF16),
            pltpu.VMEM((N_DEV, CHUNK, HALF), BF16),
            pltpu.VMEM((N_TOK, E_LOCAL), F32),
            pltpu.SemaphoreType.DMA((n_hops,)),
            pltpu.SemaphoreType.DMA((n_hops,)),
            pltpu.SemaphoreType.DMA((n_hops,)),
            pltpu.SemaphoreType.DMA((n_hops,)),
        ],
        compiler_params=pltpu.CompilerParams(collective_id=0),
    )(x, router_W, route_idx, expert_W)


# device time: 65140 ns/iter; 2.7426x vs baseline; 1.1177x over previous
import jax
import jax.numpy as jnp
from jax import lax
from jax.experimental import pallas as pl
from jax.experimental.pallas import tpu as pltpu

N_DEV = 4
N_EXP = 16
E_LOCAL = 4
N_TOK = 2048
D_IN = 512
D_OUT = 1024
CHUNK = N_TOK // N_DEV
HALF = D_OUT // 2
N_SUB = 2
SUB = CHUNK // N_SUB
BF16 = jnp.bfloat16
F32 = jnp.float32


def kernel(x, router_W, route_idx, expert_W):
    def body(x_ref, rw_ref, idx_ref, ew_ref, out_ref,
             sbuf_cw, rbuf_cw, agbuf_cw, sbuf_ccw, rbuf_ccw, agbuf_ccw,
             w_ref, ss_cw, rs_cw, ss_ccw, rs_ccw):
        p = lax.axis_index("i")
        left = lax.rem(p + N_DEV - 1, N_DEV)
        right = lax.rem(p + 1, N_DEV)

        barrier_sem = pltpu.get_barrier_semaphore()
        for nbr in (left, right):
            pl.semaphore_signal(
                barrier_sem, inc=1,
                device_id=(nbr,), device_id_type=pl.DeviceIdType.MESH,
            )
        pl.semaphore_wait(barrier_sem, 2)

        scores = jnp.dot(x_ref[:, :], rw_ref[:, :],
                         preferred_element_type=F32)
        m = jnp.max(scores, axis=1, keepdims=True)
        ex = jnp.exp(scores - m)
        probs = ex / jnp.sum(ex, axis=1, keepdims=True)
        idx0 = idx_ref[:, 0:1]
        idx1 = idx_ref[:, 1:2]
        iota = lax.broadcasted_iota(jnp.int32, (N_TOK, N_EXP), 1)
        g0 = jnp.sum(jnp.where(iota == idx0, probs, 0.0), axis=1, keepdims=True)
        g1 = jnp.sum(jnp.where(iota == idx1, probs, 0.0), axis=1, keepdims=True)
        gs = g0 + g1
        g0 = g0 / gs
        g1 = g1 / gs
        for j in range(E_LOCAL):
            eid = p * E_LOCAL + j
            w_ref[:, j:j + 1] = (jnp.where(idx0 == eid, g0, 0.0)
                                 + jnp.where(idx1 == eid, g1, 0.0))

        def rs_rdma(h, s, dirn):
            sbuf, rbuf, ss, rs, dev = (
                (sbuf_cw, rbuf_cw, ss_cw, rs_cw, right) if dirn == 0
                else (sbuf_ccw, rbuf_ccw, ss_ccw, rs_ccw, left))
            k = h * N_SUB + s
            return pltpu.make_async_remote_copy(
                src_ref=sbuf.at[h, s], dst_ref=rbuf.at[h, s],
                send_sem=ss.at[k], recv_sem=rs.at[k],
                device_id=(dev,), device_id_type=pl.DeviceIdType.MESH,
            )

        def ag_rdma(h, s, dirn):
            agbuf, ss, rs, dev = (
                (agbuf_cw, ss_cw, rs_cw, right) if dirn == 0
                else (agbuf_ccw, ss_ccw, rs_ccw, left))
            k = (N_DEV - 1 + h) * N_SUB + s
            return pltpu.make_async_remote_copy(
                src_ref=agbuf.at[h, s], dst_ref=agbuf.at[h + 1, s],
                send_sem=ss.at[k], recv_sem=rs.at[k],
                device_id=(dev,), device_id_type=pl.DeviceIdType.MESH,
            )

        hop0 = []
        for o in range(N_DEV):
            c = lax.rem(p + o, N_DEV)
            rows = pl.ds(c * CHUNK, CHUNK)
            xa = x_ref[rows, :]
            acc = None
            for j in range(E_LOCAL):
                t = jnp.dot(xa * w_ref[rows, j:j + 1], ew_ref[j],
                            preferred_element_type=F32)
                acc = t if acc is None else acc + t
            if o == 0:
                for s in range(N_SUB):
                    r0, r1 = s * SUB, (s + 1) * SUB
                    sbuf_cw[0, s] = acc[r0:r1, 0:HALF].astype(BF16)
                    sbuf_ccw[0, s] = acc[r0:r1, HALF:D_OUT].astype(BF16)
                    for dirn in (0, 1):
                        rd = rs_rdma(0, s, dirn)
                        rd.start()
                        hop0.append(rd)
            else:
                out_ref[rows, :] = acc

        rs_descs = {(0, s, d): hop0[s * 2 + d] for s in range(N_SUB)
                    for d in (0, 1)}
        for h in range(N_DEV - 1):
            rc_cw = lax.rem(p - h - 1 + N_DEV, N_DEV)
            rc_ccw = lax.rem(p + h + 1, N_DEV)
            for s in range(N_SUB):
                rs_descs[(h, s, 0)].wait()
                rs_descs[(h, s, 1)].wait()
                rows_cw = pl.ds(rc_cw * CHUNK + s * SUB, SUB)
                rows_ccw = pl.ds(rc_ccw * CHUNK + s * SUB, SUB)
                s_cw = out_ref[rows_cw, 0:HALF] + rbuf_cw[h, s].astype(F32)
                s_ccw = out_ref[rows_ccw, HALF:D_OUT] + rbuf_ccw[h, s].astype(F32)
                if h < N_DEV - 2:
                    sbuf_cw[h + 1, s] = s_cw.astype(BF16)
                    sbuf_ccw[h + 1, s] = s_ccw.astype(BF16)
                    for dirn in (0, 1):
                        rd = rs_rdma(h + 1, s, dirn)
                        rd.start()
                        rs_descs[(h + 1, s, dirn)] = rd
                else:
                    agbuf_cw[0, s] = s_cw.astype(BF16)
                    agbuf_ccw[0, s] = s_ccw.astype(BF16)
                    for dirn in (0, 1):
                        rd = ag_rdma(0, s, dirn)
                        rd.start()
                        rs_descs[("ag", s, dirn)] = rd
                out_ref[rows_cw, 0:HALF] = s_cw
                out_ref[rows_ccw, HALF:D_OUT] = s_ccw

        ag_descs = {(0, s, d): rs_descs[("ag", s, d)] for s in range(N_SUB)
                    for d in (0, 1)}
        for h in range(N_DEV - 1):
            rc_cw = lax.rem(p - h + N_DEV, N_DEV)
            rc_ccw = lax.rem(p + h, N_DEV)
            for s in range(N_SUB):
                ag_descs[(h, s, 0)].wait()
                ag_descs[(h, s, 1)].wait()
                if h < N_DEV - 2:
                    for dirn in (0, 1):
                        rd = ag_rdma(h + 1, s, dirn)
                        rd.start()
                        ag_descs[(h + 1, s, dirn)] = rd
                rows_cw = pl.ds(rc_cw * CHUNK + s * SUB, SUB)
                rows_ccw = pl.ds(rc_ccw * CHUNK + s * SUB, SUB)
                out_ref[rows_cw, 0:HALF] = agbuf_cw[h + 1, s].astype(F32)
                out_ref[rows_ccw, HALF:D_OUT] = agbuf_ccw[h + 1, s].astype(F32)

    n_sems = 2 * (N_DEV - 1) * N_SUB
    return pl.pallas_call(
        body,
        out_shape=jax.ShapeDtypeStruct((N_TOK, D_OUT), F32),
        in_specs=[
            pl.BlockSpec(memory_space=pltpu.VMEM),
            pl.BlockSpec(memory_space=pltpu.VMEM),
            pl.BlockSpec(memory_space=pltpu.VMEM),
            pl.BlockSpec(memory_space=pltpu.VMEM),
        ],
        out_specs=pl.BlockSpec(memory_space=pltpu.VMEM),
        scratch_shapes=[
            pltpu.VMEM((N_DEV - 1, N_SUB, SUB, HALF), BF16),
            pltpu.VMEM((N_DEV - 1, N_SUB, SUB, HALF), BF16),
            pltpu.VMEM((N_DEV, N_SUB, SUB, HALF), BF16),
            pltpu.VMEM((N_DEV - 1, N_SUB, SUB, HALF), BF16),
            pltpu.VMEM((N_DEV - 1, N_SUB, SUB, HALF), BF16),
            pltpu.VMEM((N_DEV, N_SUB, SUB, HALF), BF16),
            pltpu.VMEM((N_TOK, E_LOCAL), F32),
            pltpu.SemaphoreType.DMA((n_sems,)),
            pltpu.SemaphoreType.DMA((n_sems,)),
            pltpu.SemaphoreType.DMA((n_sems,)),
            pltpu.SemaphoreType.DMA((n_sems,)),
        ],
        compiler_params=pltpu.CompilerParams(collective_id=0),
    )(x, router_W, route_idx, expert_W)


# device time: 60696 ns/iter; 2.9434x vs baseline; 1.0732x over previous
import jax
import jax.numpy as jnp
from jax import lax
from jax.experimental import pallas as pl
from jax.experimental.pallas import tpu as pltpu

N_DEV = 4
N_EXP = 16
E_LOCAL = 4
N_TOK = 2048
D_IN = 512
D_OUT = 1024
CHUNK = N_TOK // N_DEV
HALF = D_OUT // 2
N_SUB = 2
SUB = CHUNK // N_SUB
BF16 = jnp.bfloat16
F32 = jnp.float32


def kernel(x, router_W, route_idx, expert_W):
    def body(x_ref, rw_ref, idx_ref, ew_ref, out_ref,
             sbuf_cw, rbuf_cw, agbuf_cw, sbuf_ccw, rbuf_ccw, agbuf_ccw,
             w_ref, xb_ref, ewb_ref, ss_cw, rs_cw, ss_ccw, rs_ccw):
        p = lax.axis_index("i")
        left = lax.rem(p + N_DEV - 1, N_DEV)
        right = lax.rem(p + 1, N_DEV)

        barrier_sem = pltpu.get_barrier_semaphore()
        for nbr in (left, right):
            pl.semaphore_signal(
                barrier_sem, inc=1,
                device_id=(nbr,), device_id_type=pl.DeviceIdType.MESH,
            )
        pl.semaphore_wait(barrier_sem, 2)

        scores = jnp.dot(x_ref[:, :], rw_ref[:, :],
                         preferred_element_type=F32)
        m = jnp.max(scores, axis=1, keepdims=True)
        ex = jnp.exp(scores - m)
        probs = ex / jnp.sum(ex, axis=1, keepdims=True)
        idx0 = idx_ref[:, 0:1]
        idx1 = idx_ref[:, 1:2]
        iota = lax.broadcasted_iota(jnp.int32, (N_TOK, N_EXP), 1)
        g0 = jnp.sum(jnp.where(iota == idx0, probs, 0.0), axis=1, keepdims=True)
        g1 = jnp.sum(jnp.where(iota == idx1, probs, 0.0), axis=1, keepdims=True)
        gs = g0 + g1
        g0 = g0 / gs
        g1 = g1 / gs
        for j in range(E_LOCAL):
            eid = p * E_LOCAL + j
            w_ref[:, j:j + 1] = (jnp.where(idx0 == eid, g0, 0.0)
                                 + jnp.where(idx1 == eid, g1, 0.0)).astype(BF16)
        xb_ref[:, :] = x_ref[:, :].astype(BF16)
        for j in range(E_LOCAL):
            ewb_ref[j] = ew_ref[j].astype(BF16)

        def rs_rdma(h, s, dirn):
            sbuf, rbuf, ss, rs, dev = (
                (sbuf_cw, rbuf_cw, ss_cw, rs_cw, right) if dirn == 0
                else (sbuf_ccw, rbuf_ccw, ss_ccw, rs_ccw, left))
            k = h * N_SUB + s
            return pltpu.make_async_remote_copy(
                src_ref=sbuf.at[h, s], dst_ref=rbuf.at[h, s],
                send_sem=ss.at[k], recv_sem=rs.at[k],
                device_id=(dev,), device_id_type=pl.DeviceIdType.MESH,
            )

        def ag_rdma(h, s, dirn):
            agbuf, ss, rs, dev = (
                (agbuf_cw, ss_cw, rs_cw, right) if dirn == 0
                else (agbuf_ccw, ss_ccw, rs_ccw, left))
            k = (N_DEV - 1 + h) * N_SUB + s
            return pltpu.make_async_remote_copy(
                src_ref=agbuf.at[h, s], dst_ref=agbuf.at[h + 1, s],
                send_sem=ss.at[k], recv_sem=rs.at[k],
                device_id=(dev,), device_id_type=pl.DeviceIdType.MESH,
            )

        def compute_chunk(c, store=True):
            rows = pl.ds(c * CHUNK, CHUNK)
            xa = xb_ref[rows, :]
            acc = None
            for j in range(E_LOCAL):
                t = jnp.dot(xa * w_ref[rows, j:j + 1], ewb_ref[j],
                            preferred_element_type=F32)
                acc = t if acc is None else acc + t
            if store:
                out_ref[rows, :] = acc
            return acc

        hop0 = []
        acc = compute_chunk(p, store=False)
        for s in range(N_SUB):
            r0, r1 = s * SUB, (s + 1) * SUB
            sbuf_cw[0, s] = acc[r0:r1, 0:HALF].astype(BF16)
            sbuf_ccw[0, s] = acc[r0:r1, HALF:D_OUT].astype(BF16)
            for dirn in (0, 1):
                rd = rs_rdma(0, s, dirn)
                rd.start()
                hop0.append(rd)
        compute_chunk(lax.rem(p + 1, N_DEV))
        compute_chunk(lax.rem(p + 3, N_DEV))

        rs_descs = {(0, s, d): hop0[s * 2 + d] for s in range(N_SUB)
                    for d in (0, 1)}
        for h in range(N_DEV - 1):
            if h == 1:
                compute_chunk(lax.rem(p + 2, N_DEV))
            rc_cw = lax.rem(p - h - 1 + N_DEV, N_DEV)
            rc_ccw = lax.rem(p + h + 1, N_DEV)
            for s in range(N_SUB):
                rs_descs[(h, s, 0)].wait()
                rs_descs[(h, s, 1)].wait()
                rows_cw = pl.ds(rc_cw * CHUNK + s * SUB, SUB)
                rows_ccw = pl.ds(rc_ccw * CHUNK + s * SUB, SUB)
                s_cw = out_ref[rows_cw, 0:HALF] + rbuf_cw[h, s].astype(F32)
                s_ccw = out_ref[rows_ccw, HALF:D_OUT] + rbuf_ccw[h, s].astype(F32)
                if h < N_DEV - 2:
                    sbuf_cw[h + 1, s] = s_cw.astype(BF16)
                    sbuf_ccw[h + 1, s] = s_ccw.astype(BF16)
                    for dirn in (0, 1):
                        rd = rs_rdma(h + 1, s, dirn)
                        rd.start()
                        rs_descs[(h + 1, s, dirn)] = rd
                else:
                    agbuf_cw[0, s] = s_cw.astype(BF16)
                    agbuf_ccw[0, s] = s_ccw.astype(BF16)
                    for dirn in (0, 1):
                        rd = ag_rdma(0, s, dirn)
                        rd.start()
                        rs_descs[("ag", s, dirn)] = rd
                out_ref[rows_cw, 0:HALF] = s_cw
                out_ref[rows_ccw, HALF:D_OUT] = s_ccw

        ag_descs = {(0, s, d): rs_descs[("ag", s, d)] for s in range(N_SUB)
                    for d in (0, 1)}
        for h in range(N_DEV - 1):
            rc_cw = lax.rem(p - h + N_DEV, N_DEV)
            rc_ccw = lax.rem(p + h, N_DEV)
            for s in range(N_SUB):
                ag_descs[(h, s, 0)].wait()
                ag_descs[(h, s, 1)].wait()
                if h < N_DEV - 2:
                    for dirn in (0, 1):
                        rd = ag_rdma(h + 1, s, dirn)
                        rd.start()
                        ag_descs[(h + 1, s, dirn)] = rd
                rows_cw = pl.ds(rc_cw * CHUNK + s * SUB, SUB)
                rows_ccw = pl.ds(rc_ccw * CHUNK + s * SUB, SUB)
                out_ref[rows_cw, 0:HALF] = agbuf_cw[h + 1, s].astype(F32)
                out_ref[rows_ccw, HALF:D_OUT] = agbuf_ccw[h + 1, s].astype(F32)

    n_sems = 2 * (N_DEV - 1) * N_SUB
    return pl.pallas_call(
        body,
        out_shape=jax.ShapeDtypeStruct((N_TOK, D_OUT), F32),
        in_specs=[
            pl.BlockSpec(memory_space=pltpu.VMEM),
            pl.BlockSpec(memory_space=pltpu.VMEM),
            pl.BlockSpec(memory_space=pltpu.VMEM),
            pl.BlockSpec(memory_space=pltpu.VMEM),
        ],
        out_specs=pl.BlockSpec(memory_space=pltpu.VMEM),
        scratch_shapes=[
            pltpu.VMEM((N_DEV - 1, N_SUB, SUB, HALF), BF16),
            pltpu.VMEM((N_DEV - 1, N_SUB, SUB, HALF), BF16),
            pltpu.VMEM((N_DEV, N_SUB, SUB, HALF), BF16),
            pltpu.VMEM((N_DEV - 1, N_SUB, SUB, HALF), BF16),
            pltpu.VMEM((N_DEV - 1, N_SUB, SUB, HALF), BF16),
            pltpu.VMEM((N_DEV, N_SUB, SUB, HALF), BF16),
            pltpu.VMEM((N_TOK, E_LOCAL), BF16),
            pltpu.VMEM((N_TOK, D_IN), BF16),
            pltpu.VMEM((E_LOCAL, D_IN, D_OUT), BF16),
            pltpu.SemaphoreType.DMA((n_sems,)),
            pltpu.SemaphoreType.DMA((n_sems,)),
            pltpu.SemaphoreType.DMA((n_sems,)),
            pltpu.SemaphoreType.DMA((n_sems,)),
        ],
        compiler_params=pltpu.CompilerParams(collective_id=0),
    )(x, router_W, route_idx, expert_W)


# device time: 57756 ns/iter; 3.0933x vs baseline; 1.0509x over previous
import jax
import jax.numpy as jnp
from jax import lax
from jax.experimental import pallas as pl
from jax.experimental.pallas import tpu as pltpu

N_DEV = 4
N_EXP = 16
E_LOCAL = 4
N_TOK = 2048
D_IN = 512
D_OUT = 1024
CHUNK = N_TOK // N_DEV
HALF = D_OUT // 2
N_SUB = 4
SUB = CHUNK // N_SUB
BF16 = jnp.bfloat16
F32 = jnp.float32


def kernel(x, router_W, route_idx, expert_W):
    def body(x_ref, rw_ref, idx_ref, ew_ref, out_ref,
             sbuf_cw, rbuf_cw, agbuf_cw, sbuf_ccw, rbuf_ccw, agbuf_ccw,
             w_ref, xb_ref, ewb_ref, ss_cw, rs_cw, ss_ccw, rs_ccw):
        p = lax.axis_index("i")
        left = lax.rem(p + N_DEV - 1, N_DEV)
        right = lax.rem(p + 1, N_DEV)

        barrier_sem = pltpu.get_barrier_semaphore()
        for nbr in (left, right):
            pl.semaphore_signal(
                barrier_sem, inc=1,
                device_id=(nbr,), device_id_type=pl.DeviceIdType.MESH,
            )
        pl.semaphore_wait(barrier_sem, 2)

        scores = jnp.dot(x_ref[:, :], rw_ref[:, :],
                         preferred_element_type=F32)
        m = jnp.max(scores, axis=1, keepdims=True)
        ex = jnp.exp(scores - m)
        probs = ex / jnp.sum(ex, axis=1, keepdims=True)
        idx0 = idx_ref[:, 0:1]
        idx1 = idx_ref[:, 1:2]
        iota = lax.broadcasted_iota(jnp.int32, (N_TOK, N_EXP), 1)
        g0 = jnp.sum(jnp.where(iota == idx0, probs, 0.0), axis=1, keepdims=True)
        g1 = jnp.sum(jnp.where(iota == idx1, probs, 0.0), axis=1, keepdims=True)
        gs = g0 + g1
        g0 = g0 / gs
        g1 = g1 / gs
        eids = (p * E_LOCAL
                + lax.broadcasted_iota(jnp.int32, (N_TOK, E_LOCAL), 1))
        w_ref[:, :] = (jnp.where(idx0 == eids, g0, 0.0)
                       + jnp.where(idx1 == eids, g1, 0.0)).astype(BF16)
        xb_ref[:, :] = x_ref[:, :].astype(BF16)
        for j in range(E_LOCAL):
            ewb_ref[j] = ew_ref[j].astype(BF16)

        def rs_rdma(h, s, dirn):
            sbuf, rbuf, ss, rs, dev = (
                (sbuf_cw, rbuf_cw, ss_cw, rs_cw, right) if dirn == 0
                else (sbuf_ccw, rbuf_ccw, ss_ccw, rs_ccw, left))
            k = h * N_SUB + s
            return pltpu.make_async_remote_copy(
                src_ref=sbuf.at[h, s], dst_ref=rbuf.at[h, s],
                send_sem=ss.at[k], recv_sem=rs.at[k],
                device_id=(dev,), device_id_type=pl.DeviceIdType.MESH,
            )

        def ag_rdma(h, s, dirn):
            agbuf, ss, rs, dev = (
                (agbuf_cw, ss_cw, rs_cw, right) if dirn == 0
                else (agbuf_ccw, ss_ccw, rs_ccw, left))
            k = (N_DEV - 1 + h) * N_SUB + s
            return pltpu.make_async_remote_copy(
                src_ref=agbuf.at[h, s], dst_ref=agbuf.at[h + 1, s],
                send_sem=ss.at[k], recv_sem=rs.at[k],
                device_id=(dev,), device_id_type=pl.DeviceIdType.MESH,
            )

        def compute_chunk(c, store=True):
            rows = pl.ds(c * CHUNK, CHUNK)
            xa = xb_ref[rows, :]
            acc = None
            for j in range(E_LOCAL):
                t = jnp.dot(xa * w_ref[rows, j:j + 1], ewb_ref[j],
                            preferred_element_type=F32)
                acc = t if acc is None else acc + t
            if store:
                out_ref[rows, :] = acc
            return acc

        hop0 = []
        acc = compute_chunk(p, store=False)
        for s in range(N_SUB):
            r0, r1 = s * SUB, (s + 1) * SUB
            sbuf_cw[0, s] = acc[r0:r1, 0:HALF].astype(BF16)
            sbuf_ccw[0, s] = acc[r0:r1, HALF:D_OUT].astype(BF16)
            for dirn in (0, 1):
                rd = rs_rdma(0, s, dirn)
                rd.start()
                hop0.append(rd)
        compute_chunk(lax.rem(p + 1, N_DEV))
        compute_chunk(lax.rem(p + 3, N_DEV))

        rs_descs = {(0, s, d): hop0[s * 2 + d] for s in range(N_SUB)
                    for d in (0, 1)}
        for h in range(N_DEV - 1):
            if h == 1:
                compute_chunk(lax.rem(p + 2, N_DEV))
            rc_cw = lax.rem(p - h - 1 + N_DEV, N_DEV)
            rc_ccw = lax.rem(p + h + 1, N_DEV)
            for s in range(N_SUB):
                for dirn in (0, 1):
                    rc = rc_cw if dirn == 0 else rc_ccw
                    cols = slice(0, HALF) if dirn == 0 else slice(HALF, D_OUT)
                    rbuf = rbuf_cw if dirn == 0 else rbuf_ccw
                    sbuf = sbuf_cw if dirn == 0 else sbuf_ccw
                    agbuf = agbuf_cw if dirn == 0 else agbuf_ccw
                    rs_descs[(h, s, dirn)].wait()
                    rows = pl.ds(rc * CHUNK + s * SUB, SUB)
                    ssum = out_ref[rows, cols] + rbuf[h, s].astype(F32)
                    if h < N_DEV - 2:
                        sbuf[h + 1, s] = ssum.astype(BF16)
                        rd = rs_rdma(h + 1, s, dirn)
                        rd.start()
                        rs_descs[(h + 1, s, dirn)] = rd
                    else:
                        agbuf[0, s] = ssum.astype(BF16)
                        rd = ag_rdma(0, s, dirn)
                        rd.start()
                        rs_descs[("ag", s, dirn)] = rd
                    out_ref[rows, cols] = ssum

        ag_descs = {(0, s, d): rs_descs[("ag", s, d)] for s in range(N_SUB)
                    for d in (0, 1)}
        for h in range(N_DEV - 1):
            rc_cw = lax.rem(p - h + N_DEV, N_DEV)
            rc_ccw = lax.rem(p + h, N_DEV)
            for s in range(N_SUB):
                for dirn in (0, 1):
                    rc = rc_cw if dirn == 0 else rc_ccw
                    cols = slice(0, HALF) if dirn == 0 else slice(HALF, D_OUT)
                    agbuf = agbuf_cw if dirn == 0 else agbuf_ccw
                    ag_descs[(h, s, dirn)].wait()
                    if h < N_DEV - 2:
                        rd = ag_rdma(h + 1, s, dirn)
                        rd.start()
                        ag_descs[(h + 1, s, dirn)] = rd
                    rows = pl.ds(rc * CHUNK + s * SUB, SUB)
                    out_ref[rows, cols] = agbuf[h + 1, s].astype(F32)

    n_sems = 2 * (N_DEV - 1) * N_SUB
    return pl.pallas_call(
        body,
        out_shape=jax.ShapeDtypeStruct((N_TOK, D_OUT), F32),
        in_specs=[
            pl.BlockSpec(memory_space=pltpu.VMEM),
            pl.BlockSpec(memory_space=pltpu.VMEM),
            pl.BlockSpec(memory_space=pltpu.VMEM),
            pl.BlockSpec(memory_space=pltpu.VMEM),
        ],
        out_specs=pl.BlockSpec(memory_space=pltpu.VMEM),
        scratch_shapes=[
            pltpu.VMEM((N_DEV - 1, N_SUB, SUB, HALF), BF16),
            pltpu.VMEM((N_DEV - 1, N_SUB, SUB, HALF), BF16),
            pltpu.VMEM((N_DEV, N_SUB, SUB, HALF), BF16),
            pltpu.VMEM((N_DEV - 1, N_SUB, SUB, HALF), BF16),
            pltpu.VMEM((N_DEV - 1, N_SUB, SUB, HALF), BF16),
            pltpu.VMEM((N_DEV, N_SUB, SUB, HALF), BF16),
            pltpu.VMEM((N_TOK, E_LOCAL), BF16),
            pltpu.VMEM((N_TOK, D_IN), BF16),
            pltpu.VMEM((E_LOCAL, D_IN, D_OUT), BF16),
            pltpu.SemaphoreType.DMA((n_sems,)),
            pltpu.SemaphoreType.DMA((n_sems,)),
            pltpu.SemaphoreType.DMA((n_sems,)),
            pltpu.SemaphoreType.DMA((n_sems,)),
        ],
        compiler_params=pltpu.CompilerParams(collective_id=0),
    )(x, router_W, route_idx, expert_W)
